# baseline (speedup 1.0000x reference)
import functools

import jax
import jax.numpy as jnp
from jax import lax
from jax.experimental import pallas as pl
from jax.experimental.pallas import tpu as pltpu
from jax.experimental.pallas import tpu_sc as plsc

ROWS, COLS = 16384, 256
L = 16
GROUPS = COLS // L
CHUNK = 64


def kernel(x, permutation):
    info = plsc.get_sparse_core_info()
    nc, ns = info.num_cores, info.num_subcores
    nw = nc * ns
    rows_per_w = ROWS // nw
    nchunk = rows_per_w // CHUNK

    mesh = plsc.VectorSubcoreMesh(core_axis_name="c", subcore_axis_name="s")

    @functools.partial(
        pl.kernel,
        mesh=mesh,
        out_type=jax.ShapeDtypeStruct((ROWS, COLS), jnp.float32),
        scratch_types=[
            pltpu.VMEM((COLS,), jnp.int32),
            pltpu.VMEM((CHUNK, COLS), jnp.float32),
            pltpu.VMEM((CHUNK, COLS), jnp.float32),
            pltpu.VMEM((CHUNK, COLS), jnp.float32),
            pltpu.VMEM((CHUNK, COLS), jnp.float32),
            pltpu.SemaphoreType.DMA,
            pltpu.SemaphoreType.DMA,
            pltpu.SemaphoreType.DMA,
            pltpu.SemaphoreType.DMA,
        ],
        compiler_params=pltpu.CompilerParams(needs_layout_passes=False),
    )
    def run(x_hbm, perm_hbm, out_hbm, perm_v, in0, in1, out0, out1,
            si0, si1, so0, so1):
        wid = lax.axis_index("s") * nc + lax.axis_index("c")
        base = wid * rows_per_w
        pltpu.sync_copy(perm_hbm, perm_v)
        perm_vecs = [perm_v[pl.ds(g * L, L)] for g in range(GROUPS)]

        ins, outs = [in0, in1], [out0, out1]
        sis, sos = [si0, si1], [so0, so1]

        def in_slice(c):
            return x_hbm.at[pl.ds(base + c * CHUNK, CHUNK)]

        def out_slice(c):
            return out_hbm.at[pl.ds(base + c * CHUNK, CHUNK)]

        def compute(in_v, out_v):
            @plsc.parallel_loop(0, CHUNK, 1, unroll=8)
            def body(r):
                row_idx = jnp.full((L,), r, dtype=jnp.int32)
                for g in range(GROUPS):
                    out_v[r, pl.ds(g * L, L)] = plsc.load_gather(
                        in_v, [row_idx, perm_vecs[g]])

        pltpu.make_async_copy(in_slice(0), ins[0], sis[0]).start()
        for c in range(nchunk):
            p = c % 2
            if c + 1 < nchunk:
                pltpu.make_async_copy(in_slice(c + 1), ins[1 - p], sis[1 - p]).start()
            pltpu.make_async_copy(in_slice(c), ins[p], sis[p]).wait()
            if c >= 2:
                pltpu.make_async_copy(outs[p], out_slice(c - 2), sos[p]).wait()
            compute(ins[p], outs[p])
            pltpu.make_async_copy(outs[p], out_slice(c), sos[p]).start()
        for c in range(max(0, nchunk - 2), nchunk):
            p = c % 2
            pltpu.make_async_copy(outs[p], out_slice(c), sos[p]).wait()

    out = run(x, permutation)
    return out, jnp.zeros(ROWS, dtype=x.dtype)


# speedup vs baseline: 1.1165x; 1.1165x over previous
import functools

import jax
import jax.numpy as jnp
from jax import lax
from jax.experimental import pallas as pl
from jax.experimental.pallas import tpu as pltpu
from jax.experimental.pallas import tpu_sc as plsc

ROWS, COLS = 16384, 256
L = 16
GROUPS = COLS // L
CHUNK = 32


def kernel(x, permutation):
    info = plsc.get_sparse_core_info()
    nc, ns = info.num_cores, info.num_subcores
    nw = nc * ns
    rows_per_w = ROWS // nw
    nchunk = rows_per_w // CHUNK

    mesh = plsc.VectorSubcoreMesh(core_axis_name="c", subcore_axis_name="s")

    @functools.partial(
        pl.kernel,
        mesh=mesh,
        out_type=jax.ShapeDtypeStruct((ROWS, COLS), jnp.float32),
        scratch_types=[
            pltpu.VMEM((COLS,), jnp.int32),
            pltpu.VMEM((CHUNK, COLS), jnp.float32),
            pltpu.VMEM((CHUNK, COLS), jnp.float32),
            pltpu.VMEM((CHUNK, COLS), jnp.float32),
            pltpu.VMEM((CHUNK, COLS), jnp.float32),
            pltpu.SemaphoreType.DMA,
            pltpu.SemaphoreType.DMA,
            pltpu.SemaphoreType.DMA,
            pltpu.SemaphoreType.DMA,
        ],
        compiler_params=pltpu.CompilerParams(needs_layout_passes=False),
    )
    def run(x_hbm, perm_hbm, out_hbm, perm_v, in0, in1, out0, out1,
            si0, si1, so0, so1):
        wid = lax.axis_index("s") * nc + lax.axis_index("c")
        base = wid * rows_per_w
        pltpu.sync_copy(perm_hbm, perm_v)
        perm_vecs = [perm_v[pl.ds(g * L, L)] for g in range(GROUPS)]

        ins, outs = [in0, in1], [out0, out1]
        sis, sos = [si0, si1], [so0, so1]

        def in_slice(c):
            return x_hbm.at[pl.ds(base + c * CHUNK, CHUNK)]

        def out_slice(c):
            return out_hbm.at[pl.ds(base + c * CHUNK, CHUNK)]

        def compute(in_v, out_v):
            @plsc.parallel_loop(0, CHUNK, 1, unroll=4)
            def body(r):
                row_idx = jnp.full((L,), r, dtype=jnp.int32)
                for g in range(GROUPS):
                    out_v[r, pl.ds(g * L, L)] = plsc.load_gather(
                        in_v, [row_idx, perm_vecs[g]])

        pltpu.make_async_copy(in_slice(0), ins[0], sis[0]).start()
        for c in range(nchunk):
            p = c % 2
            if c + 1 < nchunk:
                pltpu.make_async_copy(in_slice(c + 1), ins[1 - p], sis[1 - p]).start()
            pltpu.make_async_copy(in_slice(c), ins[p], sis[p]).wait()
            if c >= 2:
                pltpu.make_async_copy(outs[p], out_slice(c - 2), sos[p]).wait()
            compute(ins[p], outs[p])
            pltpu.make_async_copy(outs[p], out_slice(c), sos[p]).start()
        for c in range(max(0, nchunk - 2), nchunk):
            p = c % 2
            pltpu.make_async_copy(outs[p], out_slice(c), sos[p]).wait()

    out = run(x, permutation)
    return out, jnp.zeros(ROWS, dtype=x.dtype)


# speedup vs baseline: 1.3510x; 1.2100x over previous
import functools

import jax
import jax.numpy as jnp
from jax import lax
from jax.experimental import pallas as pl
from jax.experimental.pallas import tpu as pltpu
from jax.experimental.pallas import tpu_sc as plsc

ROWS, COLS = 16384, 256
L = 16
GROUPS = COLS // L
CHUNK = 64


def kernel(x, permutation):
    info = plsc.get_sparse_core_info()
    nc, ns = info.num_cores, info.num_subcores
    nw = nc * ns
    rows_per_w = ROWS // nw
    nchunk = rows_per_w // CHUNK

    mesh = plsc.VectorSubcoreMesh(core_axis_name="c", subcore_axis_name="s")

    @functools.partial(
        pl.kernel,
        mesh=mesh,
        out_type=jax.ShapeDtypeStruct((ROWS, COLS), jnp.float32),
        scratch_types=[
            pltpu.VMEM((COLS,), jnp.int32),
            pltpu.VMEM((CHUNK, COLS), jnp.float32),
            pltpu.VMEM((CHUNK, COLS), jnp.float32),
            pltpu.VMEM((CHUNK, COLS), jnp.float32),
            pltpu.VMEM((CHUNK, COLS), jnp.float32),
            pltpu.SemaphoreType.DMA,
            pltpu.SemaphoreType.DMA,
            pltpu.SemaphoreType.DMA,
            pltpu.SemaphoreType.DMA,
        ],
        compiler_params=pltpu.CompilerParams(needs_layout_passes=False),
    )
    def run(x_hbm, perm_hbm, out_hbm, perm_v, in0, in1, out0, out1,
            si0, si1, so0, so1):
        wid = lax.axis_index("s") * nc + lax.axis_index("c")
        base = wid * rows_per_w
        pltpu.sync_copy(perm_hbm, perm_v)
        perm_vecs = [perm_v[pl.ds(g * L, L)] for g in range(GROUPS)]

        ins, outs = [in0, in1], [out0, out1]
        sis, sos = [si0, si1], [so0, so1]

        def in_slice(c):
            return x_hbm.at[pl.ds(base + c * CHUNK, CHUNK)]

        def out_slice(c):
            return out_hbm.at[pl.ds(base + c * CHUNK, CHUNK)]

        def compute(in_v, out_v):
            @plsc.parallel_loop(0, CHUNK, 1, unroll=2)
            def body(r):
                row_idx = jnp.full((L,), r, dtype=jnp.int32)
                for g in range(GROUPS):
                    out_v[r, pl.ds(g * L, L)] = plsc.load_gather(
                        in_v, [row_idx, perm_vecs[g]])

        pltpu.make_async_copy(in_slice(0), ins[0], sis[0]).start()
        for c in range(nchunk):
            p = c % 2
            if c + 1 < nchunk:
                pltpu.make_async_copy(in_slice(c + 1), ins[1 - p], sis[1 - p]).start()
            pltpu.make_async_copy(in_slice(c), ins[p], sis[p]).wait()
            if c >= 2:
                pltpu.make_async_copy(outs[p], out_slice(c - 2), sos[p]).wait()
            compute(ins[p], outs[p])
            pltpu.make_async_copy(outs[p], out_slice(c), sos[p]).start()
        for c in range(max(0, nchunk - 2), nchunk):
            p = c % 2
            pltpu.make_async_copy(outs[p], out_slice(c), sos[p]).wait()

    out = run(x, permutation)
    return out, jnp.zeros(ROWS, dtype=x.dtype)


# speedup vs baseline: 1.3652x; 1.0105x over previous
import functools

import jax
import jax.numpy as jnp
from jax import lax
from jax.experimental import pallas as pl
from jax.experimental.pallas import tpu as pltpu
from jax.experimental.pallas import tpu_sc as plsc

ROWS, COLS = 16384, 256
L = 16
GROUPS = COLS // L
CHUNK = 64


def kernel(x, permutation):
    info = plsc.get_sparse_core_info()
    nc, ns = info.num_cores, info.num_subcores
    nw = nc * ns
    rows_per_w = ROWS // nw
    nchunk = rows_per_w // CHUNK

    mesh = plsc.VectorSubcoreMesh(core_axis_name="c", subcore_axis_name="s")

    @functools.partial(
        pl.kernel,
        mesh=mesh,
        out_type=jax.ShapeDtypeStruct((ROWS, COLS), jnp.float32),
        scratch_types=[
            pltpu.VMEM((COLS,), jnp.int32),
            pltpu.VMEM((CHUNK, COLS), jnp.float32),
            pltpu.VMEM((CHUNK, COLS), jnp.float32),
            pltpu.VMEM((CHUNK, COLS), jnp.float32),
            pltpu.VMEM((CHUNK, COLS), jnp.float32),
            pltpu.SemaphoreType.DMA,
            pltpu.SemaphoreType.DMA,
            pltpu.SemaphoreType.DMA,
            pltpu.SemaphoreType.DMA,
        ],
        compiler_params=pltpu.CompilerParams(needs_layout_passes=False),
    )
    def run(x_hbm, perm_hbm, out_hbm, perm_v, in0, in1, out0, out1,
            si0, si1, so0, so1):
        wid = lax.axis_index("s") * nc + lax.axis_index("c")
        base = wid * rows_per_w
        pltpu.sync_copy(perm_hbm, perm_v)
        perm_vecs = [perm_v[pl.ds(g * L, L)] for g in range(GROUPS)]

        ins, outs = [in0, in1], [out0, out1]
        sis, sos = [si0, si1], [so0, so1]

        def in_slice(c):
            return x_hbm.at[pl.ds(base + c * CHUNK, CHUNK)]

        def out_slice(c):
            return out_hbm.at[pl.ds(base + c * CHUNK, CHUNK)]

        def compute(in_v, out_v):
            @plsc.parallel_loop(0, CHUNK, 1, unroll=1)
            def body(r):
                row_idx = jnp.full((L,), r, dtype=jnp.int32)
                for g in range(GROUPS):
                    out_v[r, pl.ds(g * L, L)] = plsc.load_gather(
                        in_v, [row_idx, perm_vecs[g]])

        pltpu.make_async_copy(in_slice(0), ins[0], sis[0]).start()
        for c in range(nchunk):
            p = c % 2
            if c + 1 < nchunk:
                pltpu.make_async_copy(in_slice(c + 1), ins[1 - p], sis[1 - p]).start()
            pltpu.make_async_copy(in_slice(c), ins[p], sis[p]).wait()
            if c >= 2:
                pltpu.make_async_copy(outs[p], out_slice(c - 2), sos[p]).wait()
            compute(ins[p], outs[p])
            pltpu.make_async_copy(outs[p], out_slice(c), sos[p]).start()
        for c in range(max(0, nchunk - 2), nchunk):
            p = c % 2
            pltpu.make_async_copy(outs[p], out_slice(c), sos[p]).wait()

    out = run(x, permutation)
    return out, jnp.zeros(ROWS, dtype=x.dtype)


# speedup vs baseline: 1.3659x; 1.0005x over previous
import functools

import jax
import jax.numpy as jnp
from jax import lax
from jax.experimental import pallas as pl
from jax.experimental.pallas import tpu as pltpu
from jax.experimental.pallas import tpu_sc as plsc

ROWS, COLS = 16384, 256
L = 16
GROUPS = COLS // L
CHUNK = 64


def kernel(x, permutation):
    info = plsc.get_sparse_core_info()
    nc, ns = info.num_cores, info.num_subcores
    nw = nc * ns
    rows_per_w = ROWS // nw
    nchunk = rows_per_w // CHUNK

    mesh = plsc.VectorSubcoreMesh(core_axis_name="c", subcore_axis_name="s")

    @functools.partial(
        pl.kernel,
        mesh=mesh,
        out_type=jax.ShapeDtypeStruct((ROWS, COLS), jnp.float32),
        scratch_types=[
            pltpu.VMEM((COLS,), jnp.int32),
            pltpu.VMEM((CHUNK, COLS), jnp.float32),
            pltpu.VMEM((CHUNK, COLS), jnp.float32),
            pltpu.VMEM((CHUNK, COLS), jnp.float32),
            pltpu.VMEM((CHUNK, COLS), jnp.float32),
            pltpu.SemaphoreType.DMA,
            pltpu.SemaphoreType.DMA,
            pltpu.SemaphoreType.DMA,
            pltpu.SemaphoreType.DMA,
        ],
        compiler_params=pltpu.CompilerParams(needs_layout_passes=False, disable_bounds_checks=True, disable_semaphore_checks=True, skip_device_barrier=True),
    )
    def run(x_hbm, perm_hbm, out_hbm, perm_v, in0, in1, out0, out1,
            si0, si1, so0, so1):
        wid = lax.axis_index("s") * nc + lax.axis_index("c")
        base = wid * rows_per_w
        pltpu.sync_copy(perm_hbm, perm_v)
        perm_vecs = [perm_v[pl.ds(g * L, L)] for g in range(GROUPS)]

        ins, outs = [in0, in1], [out0, out1]
        sis, sos = [si0, si1], [so0, so1]

        def in_slice(c):
            return x_hbm.at[pl.ds(base + c * CHUNK, CHUNK)]

        def out_slice(c):
            return out_hbm.at[pl.ds(base + c * CHUNK, CHUNK)]

        def compute(in_v, out_v):
            @plsc.parallel_loop(0, CHUNK, 1, unroll=1)
            def body(r):
                row_idx = jnp.full((L,), r, dtype=jnp.int32)
                for g in range(GROUPS):
                    out_v[r, pl.ds(g * L, L)] = plsc.load_gather(
                        in_v, [row_idx, perm_vecs[g]])

        pltpu.make_async_copy(in_slice(0), ins[0], sis[0]).start()
        for c in range(nchunk):
            p = c % 2
            if c + 1 < nchunk:
                pltpu.make_async_copy(in_slice(c + 1), ins[1 - p], sis[1 - p]).start()
            pltpu.make_async_copy(in_slice(c), ins[p], sis[p]).wait()
            if c >= 2:
                pltpu.make_async_copy(outs[p], out_slice(c - 2), sos[p]).wait()
            compute(ins[p], outs[p])
            pltpu.make_async_copy(outs[p], out_slice(c), sos[p]).start()
        for c in range(max(0, nchunk - 2), nchunk):
            p = c % 2
            pltpu.make_async_copy(outs[p], out_slice(c), sos[p]).wait()

    out = run(x, permutation)
    return out, jnp.zeros(ROWS, dtype=x.dtype)
